# Initial kernel scaffold; baseline (speedup 1.0000x reference)
import jax
import jax.numpy as jnp
from jax.experimental import pallas as pl


def kernel(input_, edge, W, a):
    raise NotImplementedError("write your pallas kernel here")


# speedup vs baseline: 2.8328x; 2.8328x over previous
import functools

import jax
import jax.numpy as jnp
from jax import lax
from jax.experimental import pallas as pl
from jax.experimental.pallas import tpu as pltpu
from jax.experimental.pallas import tpu_sc as plsc

N = 10000
D = 128
DP = 144
E = 320000
ALPHA = 0.2

NC, NS = 2, 16
NW = NC * NS
EPT = E // NW
SUB = 80
IBLK = 25
NBLK = EPT // (SUB * IBLK)
NP = 10112
ROWS_PT = NP // NS

_B = 1000
_GRID = N // _B


def _prep_body(x_ref, w_ref, a_ref, hs_ref, hdp_ref, s1_ref):
    x = x_ref[...]
    w = w_ref[...]
    a = a_ref[...]
    dn = (((1,), (1,)), ((), ()))
    hs = lax.dot_general(x, w[:, :D], dn, preferred_element_type=jnp.float32)
    hd = lax.dot_general(x, w[:, D:], dn, preferred_element_type=jnp.float32)
    hs_ref[...] = hs
    s2 = lax.dot_general(hd, a, dn, preferred_element_type=jnp.float32)
    hdp_ref[...] = jnp.concatenate(
        [hd, jnp.ones((_B, 1), jnp.float32), s2,
         jnp.zeros((_B, DP - D - 2), jnp.float32)], axis=1)
    s1_ref[...] = lax.dot_general(hs, a, dn, preferred_element_type=jnp.float32)


_prep = pl.pallas_call(
    _prep_body,
    grid=(_GRID,),
    in_specs=[
        pl.BlockSpec((_B, D), lambda i: (i, 0)),
        pl.BlockSpec((D, 2 * D), lambda i: (0, 0)),
        pl.BlockSpec((1, D), lambda i: (0, 0)),
    ],
    out_specs=[
        pl.BlockSpec((_B, D), lambda i: (i, 0)),
        pl.BlockSpec((_B, DP), lambda i: (i, 0)),
        pl.BlockSpec((_B, 1), lambda i: (i, 0)),
    ],
    out_shape=[
        jax.ShapeDtypeStruct((N, D), jnp.float32),
        jax.ShapeDtypeStruct((N, DP), jnp.float32),
        jax.ShapeDtypeStruct((N, 1), jnp.float32),
    ],
)


def _finish_body(hs_ref, p_ref, o_ref):
    p0 = p_ref[0]
    p1 = p_ref[1]
    acc = p0[:, :D] + p1[:, :D]
    rs = p0[:, D:D + 1] + p1[:, D:D + 1]
    denom = jnp.where(rs == 0.0, 1e-12, rs)
    h = (rs * hs_ref[...] + acc) / denom
    o_ref[...] = jnp.where(h > 0, h, jnp.exp(jnp.minimum(h, 0.0)) - 1.0)


_finish = pl.pallas_call(
    _finish_body,
    grid=(_GRID,),
    in_specs=[
        pl.BlockSpec((_B, D), lambda i: (i, 0)),
        pl.BlockSpec((NC, _B, DP), lambda i: (0, i, 0)),
    ],
    out_specs=pl.BlockSpec((_B, D), lambda i: (i, 0)),
    out_shape=jax.ShapeDtypeStruct((N, D), jnp.float32),
)


@functools.cache
def _make_sc_edges():
    return pl.kernel(
        _sc_edges_body,
        out_type=jax.ShapeDtypeStruct((NC, NP, DP), jnp.float32),
        mesh=plsc.VectorSubcoreMesh(core_axis_name="c", subcore_axis_name="s"),
        compiler_params=pltpu.CompilerParams(
            needs_layout_passes=False, use_tc_tiling_on_sc=False),
        scratch_types=[
            pltpu.VMEM((IBLK, SUB), jnp.int32),
            pltpu.VMEM((IBLK, SUB), jnp.int32),
            pltpu.VMEM((SUB, DP), jnp.float32),
            pltpu.VMEM((N,), jnp.float32),
            pltpu.VMEM_SHARED((NP, DP), jnp.float32),
            pltpu.SemaphoreType.DMA,
        ],
    )


def _sc_edges_body(hdp_hbm, e0_hbm, e1_hbm, s1_hbm, zer_hbm, out_hbm,
                   e0_v, e1_v, rows_v, s1_v, acc_sh, sem):
    cid = lax.axis_index("c")
    sid = lax.axis_index("s")
    wid = cid * NS + sid

    pltpu.sync_copy(zer_hbm, acc_sh.at[pl.ds(sid * ROWS_PT, ROWS_PT)])
    pltpu.sync_copy(s1_hbm, s1_v)
    plsc.subcore_barrier()

    def block(b, carry):
        pltpu.sync_copy(e0_hbm.at[wid, b], e0_v)
        pltpu.sync_copy(e1_hbm.at[wid, b], e1_v)

        def chunk(c, carry2):
            pltpu.async_copy(hdp_hbm.at[e1_v.at[c]], rows_v, sem).wait()
            for gi in range(SUB // 16):
                eids = lax.iota(jnp.int32, 16) + gi * 16
                e0g = e0_v[c, pl.ds(gi * 16, 16)]
                s1g = plsc.load_gather(s1_v, [e0g])
                s2g = plsc.load_gather(
                    rows_v, [eids, jnp.full((16,), D + 1, jnp.int32)])
                lg = s1g + s2g
                lr = jnp.where(lg >= 0.0, lg, ALPHA * lg)
                w = jnp.exp(-lr)
                for cc in range(D + 2):
                    col = jnp.full((16,), cc, jnp.int32)
                    v = plsc.load_gather(rows_v, [eids, col])
                    plsc.store_scatter(rows_v, [eids, col], v * w)
            pltpu.sync_copy(rows_v, acc_sh.at[e0_v.at[c]], add=True)
            return carry2

        lax.fori_loop(0, IBLK, chunk, 0)
        return carry

    lax.fori_loop(0, NBLK, block, 0)
    plsc.subcore_barrier()
    pltpu.sync_copy(acc_sh.at[pl.ds(sid * ROWS_PT, ROWS_PT)],
                    out_hbm.at[cid, pl.ds(sid * ROWS_PT, ROWS_PT)])


def kernel(input_, edge, W, a):
    e0 = edge[0].astype(jnp.int32).reshape(NW, NBLK, IBLK, SUB)
    e1 = edge[1].astype(jnp.int32).reshape(NW, NBLK, IBLK, SUB)
    hs, hdp, s1 = _prep(input_, W, a)
    zer = jnp.zeros((ROWS_PT, DP), jnp.float32)
    partials = _make_sc_edges()(hdp, e0, e1, s1.reshape(N), zer)
    return _finish(hs, partials)
